# baseline (speedup 1.0000x reference)
import functools
import jax
import jax.numpy as jnp
from jax import lax
from jax.experimental import pallas as pl
from jax.experimental.pallas import tpu as pltpu, tpu_sc as plsc

NUM_C = 2000
EMB = 128
SIZE_M = 32
NUM_STU = 2048
B = 16
L = 200
F32 = jnp.float32


def _sig(x):
    return 1.0 / (1.0 + jnp.exp(-x))


def _dot(a, b):
    return jnp.dot(a, b, preferred_element_type=F32)


def _x1_body(stu_ref, w_ref, b_ref, o_ref):
    o_ref[...] = _dot(stu_ref[...], w_ref[...]) + b_ref[...]


def _x1(stu, W1, b1row):
    return pl.pallas_call(
        _x1_body,
        out_shape=jax.ShapeDtypeStruct((NUM_STU, EMB), F32),
    )(stu, W1, b1row)


_GBLK = 256


def _gmm_epi_body(g_ref, x_ref, w2_ref, b2_ref, o_ref):
    t = _dot(g_ref[...], x_ref[...])
    o_ref[...] = _dot(jnp.maximum(t, 0.0), w2_ref[...]) + b2_ref[...]


def _gmm_body(g_ref, x_ref, o_ref):
    o_ref[...] = _dot(g_ref[...], x_ref[...])


def _gmm_epi(G, X, W2, b2row):
    n = NUM_STU // _GBLK
    return pl.pallas_call(
        _gmm_epi_body,
        grid=(n,),
        in_specs=[
            pl.BlockSpec((_GBLK, NUM_STU), lambda i: (i, 0)),
            pl.BlockSpec((NUM_STU, EMB), lambda i: (0, 0)),
            pl.BlockSpec((EMB, EMB), lambda i: (0, 0)),
            pl.BlockSpec((1, EMB), lambda i: (0, 0)),
        ],
        out_specs=pl.BlockSpec((_GBLK, EMB), lambda i: (i, 0)),
        out_shape=jax.ShapeDtypeStruct((NUM_STU, EMB), F32),
    )(G, X, W2, b2row)


def _gmm(G, X):
    n = NUM_STU // _GBLK
    return pl.pallas_call(
        _gmm_body,
        grid=(n,),
        in_specs=[
            pl.BlockSpec((_GBLK, NUM_STU), lambda i: (i, 0)),
            pl.BlockSpec((NUM_STU, EMB), lambda i: (0, 0)),
        ],
        out_specs=pl.BlockSpec((_GBLK, EMB), lambda i: (i, 0)),
        out_shape=jax.ShapeDtypeStruct((NUM_STU, EMB), F32),
    )(G, X)


_PAD = 3328
_WCOLS = 768


def _pack_bf16(x):
    n, w = x.shape
    return lax.bitcast_convert_type(
        x.astype(jnp.bfloat16).reshape(n, w // 2, 2), F32)


def _unpack_bf16(x):
    n, w = x.shape
    return lax.bitcast_convert_type(x, jnp.bfloat16).reshape(n, 2 * w)


def _sc_gather(widths, *tabs_and_idx, dtype=F32):
    n = len(widths)
    info = plsc.get_sparse_core_info()
    nc, ns = info.num_cores, info.num_subcores
    nw = nc * ns
    bpw = _PAD // nw
    mesh = plsc.VectorSubcoreMesh(core_axis_name="c", subcore_axis_name="s")

    scratch = []
    for w in widths:
        scratch.append(pltpu.VMEM((bpw,), jnp.int32))
        scratch.append(pltpu.VMEM((bpw, w), dtype))
    scratch.append(pltpu.SemaphoreType.DMA)

    @functools.partial(
        pl.kernel,
        mesh=mesh,
        out_type=[jax.ShapeDtypeStruct((_PAD, w), dtype) for w in widths],
        scratch_types=scratch,
    )
    def gather_k(*refs):
        tabs = refs[0:2 * n:2]
        idxs = refs[1:2 * n:2]
        outs = refs[2 * n:3 * n]
        ivs = refs[3 * n:3 * n + 2 * n:2]
        rvs = refs[3 * n + 1:3 * n + 2 * n:2]
        sem = refs[-1]
        wid = lax.axis_index("s") * nc + lax.axis_index("c")
        base = wid * bpw
        for th, ih, oh, iv, rv in zip(tabs, idxs, outs, ivs, rvs):
            pltpu.sync_copy(ih.at[pl.ds(base, bpw)], iv)
            pltpu.async_copy(th.at[iv], rv, sem).wait()
            pltpu.sync_copy(rv, oh.at[pl.ds(base, bpw)])

    return gather_k(*tabs_and_idx)


def _prep_body(sh_ref, k0_ref, v0_ref, scol_ref, srow_ref, sncol_ref,
               snrow_ref, m_ref, g1w_ref, g1b_ref, g2w_ref, g2b_ref,
               gkh_ref, gvh_ref, gkd_ref, gvd_ref, mkt_ref,
               ehw_ref, ehb_ref, ahw_ref, ahb_ref,
               edw_ref, edb_ref, adw_ref, adb_ref,
               kh_o, kd_o, wh_o, wd_o, eh_o, ah_o, ed_o, ad_o):
    sh = sh_ref[0]
    k0 = k0_ref[0]
    v0 = v0_ref[0]
    scol = scol_ref[0]
    srow = srow_ref[0]
    sncol = sncol_ref[0]
    snrow = snrow_ref[0]
    mcol = m_ref[0]

    emask = (snrow == scol).astype(F32)
    emask2 = (sncol == srow).astype(F32)
    deg_col = 1.0 + jnp.sum(emask, axis=1, keepdims=True)
    deg_row = 1.0 + jnp.sum(emask2, axis=0, keepdims=True)
    e_adj = emask * lax.rsqrt(deg_col) * lax.rsqrt(deg_row)
    selfco = 1.0 / deg_col
    xw1 = _dot(k0, g1w_ref[...])
    h1 = jnp.maximum(_dot(e_adj, xw1) + selfco * xw1 + g1b_ref[...], 0.0)
    xw2 = _dot(h1, g2w_ref[...])
    out2 = _dot(e_adj, xw2) + selfco * xw2 + g2b_ref[...]
    mean_h = jnp.sum(mcol * out2, axis=0, keepdims=True) * (1.0 / L)
    ash = jnp.broadcast_to(mean_h, (L, EMB))

    def gate(a, c, w_ref):
        w = w_ref[...]
        g = _sig(_dot(a, w[:EMB]) + _dot(c, w[EMB:]))
        return g * a + (1.0 - g) * c

    kh = gate(sh, k0, gkh_ref)
    vh = gate(sh, v0, gvh_ref)
    kd = gate(ash, k0, gkd_ref)
    vd = gate(ash, v0, gvd_ref)

    mkt = mkt_ref[...]

    def softmax32(x):
        m = jnp.max(x, axis=1, keepdims=True)
        ex = jnp.exp(x - m)
        return ex / jnp.sum(ex, axis=1, keepdims=True)

    kh_o[0] = kh
    kd_o[0] = kd
    wh_o[0] = softmax32(_dot(kh, mkt))
    wd_o[0] = softmax32(_dot(kd, mkt))
    eh_o[0] = _sig(_dot(vh, ehw_ref[...]) + ehb_ref[...])
    ah_o[0] = jnp.tanh(_dot(vh, ahw_ref[...]) + ahb_ref[...])
    ed_o[0] = _sig(_dot(vd, edw_ref[...]) + edb_ref[...])
    ad_o[0] = jnp.tanh(_dot(vd, adw_ref[...]) + adb_ref[...])


def _prep(sh, k0, v0, scol, srow, sncol, snrow, maskc, p, MkT):
    bl128 = pl.BlockSpec((1, L, EMB), lambda b: (b, 0, 0))
    full = lambda shape: pl.BlockSpec(shape, lambda b: tuple(0 for _ in shape))
    ins = [
        bl128, bl128, bl128,
        pl.BlockSpec((1, L, 1), lambda b: (b, 0, 0)),
        pl.BlockSpec((1, 1, L), lambda b: (b, 0, 0)),
        pl.BlockSpec((1, L, 1), lambda b: (b, 0, 0)),
        pl.BlockSpec((1, 1, L), lambda b: (b, 0, 0)),
        pl.BlockSpec((1, L, 1), lambda b: (b, 0, 0)),
        full((EMB, 8)), full((1, 8)), full((8, EMB)), full((1, EMB)),
        full((2 * EMB, 1)), full((2 * EMB, 1)), full((2 * EMB, 1)),
        full((2 * EMB, 1)), full((EMB, SIZE_M)),
        full((EMB, EMB)), full((1, EMB)), full((EMB, EMB)), full((1, EMB)),
        full((EMB, EMB)), full((1, EMB)), full((EMB, EMB)), full((1, EMB)),
    ]
    bl32 = pl.BlockSpec((1, L, SIZE_M), lambda b: (b, 0, 0))
    outs = [bl128, bl128, bl32, bl32, bl128, bl128, bl128, bl128]
    oshape = [
        jax.ShapeDtypeStruct((B, L, EMB), F32),
        jax.ShapeDtypeStruct((B, L, EMB), F32),
        jax.ShapeDtypeStruct((B, L, SIZE_M), F32),
        jax.ShapeDtypeStruct((B, L, SIZE_M), F32),
        jax.ShapeDtypeStruct((B, L, EMB), F32),
        jax.ShapeDtypeStruct((B, L, EMB), F32),
        jax.ShapeDtypeStruct((B, L, EMB), F32),
        jax.ShapeDtypeStruct((B, L, EMB), F32),
    ]
    r1 = lambda v: v.reshape(1, -1)
    return pl.pallas_call(
        _prep_body,
        grid=(B,),
        in_specs=ins,
        out_specs=outs,
        out_shape=oshape,
    )(sh, k0, v0, scol, srow, sncol, snrow, maskc,
      p['g1_W'], r1(p['g1_b']), p['g2_W'], r1(p['g2_b']),
      p['gkh_W'], p['gvh_W'], p['gkd_W'], p['gvd_W'], MkT,
      p['eh_W'], r1(p['eh_b']), p['ah_W'], r1(p['ah_b']),
      p['ed_W'], r1(p['ed_b']), p['ad_W'], r1(p['ad_b']))


_TCH = 8


def _scan_body(mv0_ref, wh_ref, eh_ref, ah_ref, wd_ref, ed_ref, ad_ref,
               fh_ref, fd_ref, mv_s):
    @pl.when(pl.program_id(0) == 0)
    def _init():
        mv_s[...] = jnp.broadcast_to(mv0_ref[...][None, None],
                                     (2, B, SIZE_M, EMB))

    for j in range(_TCH):
        for idx, (w_ref, e_ref, a_ref, f_ref) in enumerate(
                ((wh_ref, eh_ref, ah_ref, fh_ref),
                 (wd_ref, ed_ref, ad_ref, fd_ref))):
            mv = mv_s[idx]
            wt = w_ref[:, j, :]
            et = e_ref[:, j, :]
            at = a_ref[:, j, :]
            pmat = mv * wt[:, :, None]
            f_ref[:, j, :] = jnp.sum(pmat, axis=1)
            mv_s[idx] = mv - pmat * et[:, None, :] + wt[:, :, None] * at[:, None, :]


def _scan(Mv0, wh, eh, ah, wd, ed, ad):
    n = L // _TCH
    b32 = pl.BlockSpec((B, _TCH, SIZE_M), lambda i: (0, i, 0))
    b128 = pl.BlockSpec((B, _TCH, EMB), lambda i: (0, i, 0))
    return pl.pallas_call(
        _scan_body,
        grid=(n,),
        in_specs=[pl.BlockSpec((SIZE_M, EMB), lambda i: (0, 0)),
                  b32, b128, b128, b32, b128, b128],
        out_specs=[b128, b128],
        out_shape=[jax.ShapeDtypeStruct((B, L, EMB), F32),
                   jax.ShapeDtypeStruct((B, L, EMB), F32)],
        scratch_shapes=[pltpu.VMEM((2, B, SIZE_M, EMB), F32)],
    )(Mv0, wh, eh, ah, wd, ed, ad)


def _final_body(fh_ref, fd_ref, kh_ref, kd_ref, wg_ref,
                fhw_ref, fhb_ref, fdw_ref, fdb_ref,
                w1_ref, w2_ref, wb_ref,
                ph_o, pd_o, pe_o):
    fh = fh_ref[0]
    fd = fd_ref[0]
    kh = kh_ref[0]
    kd = kd_ref[0]
    fhw = fhw_ref[...]
    fdw = fdw_ref[...]
    h = jnp.tanh(_dot(fh, fhw[:EMB]) + _dot(kh, fhw[EMB:]) + fhb_ref[...])
    d = jnp.tanh(_dot(fd, fdw[:EMB]) + _dot(kd, fdw[EMB:]) + fdb_ref[...])
    th = _sig(_dot(h, w1_ref[...]) + _dot(d, w2_ref[...]) + wb_ref[...])
    h2 = th * h
    d2 = (1.0 - th) * d
    wg = wg_ref[0]
    hc = h[:L - 1]
    dc = d[:L - 1]
    h2c = h2[:L - 1]
    d2c = d2[:L - 1]
    ph_o[0] = jnp.sum(hc * wg[:, :EMB], axis=1, keepdims=True) \
        + wg[:, 4 * EMB:4 * EMB + 1]
    pd_o[0] = jnp.sum(dc * wg[:, EMB:2 * EMB], axis=1, keepdims=True) \
        + wg[:, 4 * EMB + 1:4 * EMB + 2]
    pe_o[0] = jnp.sum(h2c * wg[:, 2 * EMB:3 * EMB], axis=1, keepdims=True) \
        + jnp.sum(d2c * wg[:, 3 * EMB:4 * EMB], axis=1, keepdims=True) \
        + wg[:, 4 * EMB + 2:4 * EMB + 3]


def _final(fh, fd, kh, kd, Wg, p, wbrow):
    bl128 = pl.BlockSpec((1, L, EMB), lambda b: (b, 0, 0))
    full = lambda shape: pl.BlockSpec(shape, lambda b: tuple(0 for _ in shape))
    r1 = lambda v: v.reshape(1, -1)
    out1 = pl.BlockSpec((1, L - 1, 1), lambda b: (b, 0, 0))
    osh = jax.ShapeDtypeStruct((B, L - 1, 1), F32)
    return pl.pallas_call(
        _final_body,
        grid=(B,),
        in_specs=[bl128, bl128, bl128, bl128,
                  pl.BlockSpec((1, L - 1, _WCOLS), lambda b: (b, 0, 0)),
                  full((2 * EMB, EMB)), full((1, EMB)),
                  full((2 * EMB, EMB)), full((1, EMB)),
                  full((EMB, EMB)), full((EMB, EMB)), full((1, EMB))],
        out_specs=[out1, out1, out1],
        out_shape=[osh, osh, osh],
    )(fh, fd, kh, kd, Wg,
      p['fh_W'], r1(p['fh_b']), p['fd_W'], r1(p['fd_b']),
      p['w1_W'], p['w2_W'], wbrow)


def kernel(params, G, student, skill, answer):
    p = params

    answer_x = jnp.where(answer == 2, 1, answer)
    x_idx = skill + NUM_C * answer_x
    pad = lambda v: jnp.concatenate(
        [v.ravel(), jnp.zeros((_PAD - v.size,), jnp.int32)])
    idx_stu = pad(student - 1)
    idx_v = pad(x_idx)
    idx_w = pad(skill[:, 1:])

    bias3 = jnp.stack([p['h_b'], p['d_b'], p['ens_b']], 1)
    Wcat = jnp.concatenate(
        [p['h_W'].T, p['d_W'].T, p['ens_W'].T, bias3,
         jnp.zeros((NUM_C, _WCOLS - 4 * EMB - 3), F32)], 1)
    Wpack = _pack_bf16(Wcat)
    kv_top = jnp.concatenate(
        [p['k_emb'][:NUM_C], p['k_emb'][:NUM_C], p['k_emb'][:1]], 0)
    kv_pack = _pack_bf16(jnp.concatenate([kv_top, p['v_emb']], 1))

    sf = skill.astype(F32)
    snf = jnp.concatenate([sf[:, 1:], jnp.full((B, 1), -1.0, F32)], 1)
    maskf = (answer != 2).astype(F32)
    scol = sf.reshape(B, L, 1)
    srow = sf.reshape(B, 1, L)
    sncol = snf.reshape(B, L, 1)
    snrow = snf.reshape(B, 1, L)
    maskc = maskf.reshape(B, L, 1)
    MkT = p['Mk'].T
    wbrow = (p['w1_b'] + p['w2_b']).reshape(1, EMB)

    g_kv, g_w = _sc_gather(
        (2 * EMB // 2, _WCOLS // 2), kv_pack, idx_v, Wpack, idx_w)

    X1 = _x1(p['stu'], p['hg_W1'], p['hg_b1'].reshape(1, EMB))
    X3 = _gmm_epi(G, X1, p['hg_W2'], p['hg_b2'].reshape(1, EMB))
    stu_emb = _gmm(G, X3)

    (g_stu,) = _sc_gather((EMB,), stu_emb, idx_stu)
    stu_h = g_stu[:B * L].reshape(B, L, EMB)
    kv = _unpack_bf16(g_kv)[:B * L].astype(F32)
    k0 = kv[:, :EMB].reshape(B, L, EMB)
    v0 = kv[:, EMB:].reshape(B, L, EMB)
    Wg = _unpack_bf16(g_w)[:B * (L - 1)].astype(F32).reshape(
        B, L - 1, _WCOLS)

    kh, kd, wh, wd, eh, ah, ed, ad = _prep(
        stu_h, k0, v0, scol, srow, sncol, snrow, maskc, p, MkT)

    fh, fd = _scan(p['Mv0'], wh, eh, ah, wd, ed, ad)

    ph, pd, pe = _final(fh, fd, kh, kd, Wg, p, wbrow)
    return ph[..., 0], pd[..., 0], pe[..., 0]


# speedup vs baseline: 1.5907x; 1.5907x over previous
import functools
import jax
import jax.numpy as jnp
from jax import lax
from jax.experimental import pallas as pl
from jax.experimental.pallas import tpu as pltpu, tpu_sc as plsc

NUM_C = 2000
EMB = 128
SIZE_M = 32
NUM_STU = 2048
B = 16
L = 200
F32 = jnp.float32


def _sig(x):
    return 1.0 / (1.0 + jnp.exp(-x))


def _dot(a, b):
    return jnp.dot(a, b, preferred_element_type=F32)


def _x1_body(stu_ref, w_ref, b_ref, o_ref):
    o_ref[...] = _dot(stu_ref[...], w_ref[...]) + b_ref[...]


def _x1(stu, W1, b1row):
    return pl.pallas_call(
        _x1_body,
        out_shape=jax.ShapeDtypeStruct((NUM_STU, EMB), F32),
    )(stu, W1, b1row)


_GBLK = 256


def _gmm_epi_body(g_ref, x_ref, w2_ref, b2_ref, o_ref):
    t = _dot(g_ref[...], x_ref[...])
    o_ref[...] = _dot(jnp.maximum(t, 0.0), w2_ref[...]) + b2_ref[...]


def _gmm_body(g_ref, x_ref, o_ref):
    o_ref[...] = _dot(g_ref[...], x_ref[...])


def _gmm_epi(G, X, W2, b2row):
    n = NUM_STU // _GBLK
    return pl.pallas_call(
        _gmm_epi_body,
        grid=(n,),
        in_specs=[
            pl.BlockSpec((_GBLK, NUM_STU), lambda i: (i, 0)),
            pl.BlockSpec((NUM_STU, EMB), lambda i: (0, 0)),
            pl.BlockSpec((EMB, EMB), lambda i: (0, 0)),
            pl.BlockSpec((1, EMB), lambda i: (0, 0)),
        ],
        out_specs=pl.BlockSpec((_GBLK, EMB), lambda i: (i, 0)),
        out_shape=jax.ShapeDtypeStruct((NUM_STU, EMB), F32),
    )(G, X, W2, b2row)


def _gmm(G, X):
    n = NUM_STU // _GBLK
    return pl.pallas_call(
        _gmm_body,
        grid=(n,),
        in_specs=[
            pl.BlockSpec((_GBLK, NUM_STU), lambda i: (i, 0)),
            pl.BlockSpec((NUM_STU, EMB), lambda i: (0, 0)),
        ],
        out_specs=pl.BlockSpec((_GBLK, EMB), lambda i: (i, 0)),
        out_shape=jax.ShapeDtypeStruct((NUM_STU, EMB), F32),
    )(G, X)


_PAD = 3328
_WCOLS = 384


def _packhl(a, b):
    au = lax.bitcast_convert_type(a.astype(jnp.bfloat16), jnp.uint16)
    bu = lax.bitcast_convert_type(b.astype(jnp.bfloat16), jnp.uint16)
    w = jnp.left_shift(au.astype(jnp.uint32), 16) | bu.astype(jnp.uint32)
    return lax.bitcast_convert_type(w, F32)


def _hi(w):
    wu = lax.bitcast_convert_type(w, jnp.int32)
    return lax.bitcast_convert_type(wu & -65536, F32)


def _lo(w):
    wu = lax.bitcast_convert_type(w, jnp.int32)
    return lax.bitcast_convert_type(jnp.left_shift(wu, 16), F32)


def _sc_gather(widths, *tabs_and_idx, dtype=F32):
    n = len(widths)
    info = plsc.get_sparse_core_info()
    nc, ns = info.num_cores, info.num_subcores
    nw = nc * ns
    bpw = _PAD // nw
    mesh = plsc.VectorSubcoreMesh(core_axis_name="c", subcore_axis_name="s")

    scratch = []
    for w in widths:
        scratch.append(pltpu.VMEM((bpw,), jnp.int32))
        scratch.append(pltpu.VMEM((bpw, w), dtype))
    scratch.append(pltpu.SemaphoreType.DMA)

    @functools.partial(
        pl.kernel,
        mesh=mesh,
        out_type=[jax.ShapeDtypeStruct((_PAD, w), dtype) for w in widths],
        scratch_types=scratch,
    )
    def gather_k(*refs):
        tabs = refs[0:2 * n:2]
        idxs = refs[1:2 * n:2]
        outs = refs[2 * n:3 * n]
        ivs = refs[3 * n:3 * n + 2 * n:2]
        rvs = refs[3 * n + 1:3 * n + 2 * n:2]
        sem = refs[-1]
        wid = lax.axis_index("s") * nc + lax.axis_index("c")
        base = wid * bpw
        for th, ih, oh, iv, rv in zip(tabs, idxs, outs, ivs, rvs):
            pltpu.sync_copy(ih.at[pl.ds(base, bpw)], iv)
            pltpu.async_copy(th.at[iv], rv, sem).wait()
            pltpu.sync_copy(rv, oh.at[pl.ds(base, bpw)])

    return gather_k(*tabs_and_idx)


def _prep_body(sh_ref, kv_ref, scol_ref, srow_ref, sncol_ref,
               snrow_ref, m_ref, g1w_ref, g1b_ref, g2w_ref, g2b_ref,
               gkh_ref, gvh_ref, gkd_ref, gvd_ref, mkt_ref,
               ehw_ref, ehb_ref, ahw_ref, ahb_ref,
               edw_ref, edb_ref, adw_ref, adb_ref,
               kh_o, kd_o, wh_o, wd_o, eh_o, ah_o, ed_o, ad_o):
    sh = sh_ref[0]
    kvp = kv_ref[0]
    k0 = _hi(kvp)
    v0 = _lo(kvp)
    scol = scol_ref[0]
    srow = srow_ref[0]
    sncol = sncol_ref[0]
    snrow = snrow_ref[0]
    mcol = m_ref[0]

    emask = (snrow == scol).astype(F32)
    emask2 = (sncol == srow).astype(F32)
    deg_col = 1.0 + jnp.sum(emask, axis=1, keepdims=True)
    deg_row = 1.0 + jnp.sum(emask2, axis=0, keepdims=True)
    e_adj = emask * lax.rsqrt(deg_col) * lax.rsqrt(deg_row)
    selfco = 1.0 / deg_col
    xw1 = _dot(k0, g1w_ref[...])
    h1 = jnp.maximum(_dot(e_adj, xw1) + selfco * xw1 + g1b_ref[...], 0.0)
    xw2 = _dot(h1, g2w_ref[...])
    out2 = _dot(e_adj, xw2) + selfco * xw2 + g2b_ref[...]
    mean_h = jnp.sum(mcol * out2, axis=0, keepdims=True) * (1.0 / L)
    ash = jnp.broadcast_to(mean_h, (L, EMB))

    def gate(a, c, w_ref):
        w = w_ref[...]
        g = _sig(_dot(a, w[:EMB]) + _dot(c, w[EMB:]))
        return g * a + (1.0 - g) * c

    kh = gate(sh, k0, gkh_ref)
    vh = gate(sh, v0, gvh_ref)
    kd = gate(ash, k0, gkd_ref)
    vd = gate(ash, v0, gvd_ref)

    mkt = mkt_ref[...]

    def softmax32(x):
        m = jnp.max(x, axis=1, keepdims=True)
        ex = jnp.exp(x - m)
        return ex / jnp.sum(ex, axis=1, keepdims=True)

    kh_o[0] = kh
    kd_o[0] = kd
    wh_o[0] = softmax32(_dot(kh, mkt))
    wd_o[0] = softmax32(_dot(kd, mkt))
    eh_o[0] = _sig(_dot(vh, ehw_ref[...]) + ehb_ref[...])
    ah_o[0] = jnp.tanh(_dot(vh, ahw_ref[...]) + ahb_ref[...])
    ed_o[0] = _sig(_dot(vd, edw_ref[...]) + edb_ref[...])
    ad_o[0] = jnp.tanh(_dot(vd, adw_ref[...]) + adb_ref[...])


def _prep(sh, kv, scol, srow, sncol, snrow, maskc, p, MkT):
    bl128 = pl.BlockSpec((1, L, EMB), lambda b: (b, 0, 0))
    full = lambda shape: pl.BlockSpec(shape, lambda b: tuple(0 for _ in shape))
    ins = [
        bl128, bl128,
        pl.BlockSpec((1, L, 1), lambda b: (b, 0, 0)),
        pl.BlockSpec((1, 1, L), lambda b: (b, 0, 0)),
        pl.BlockSpec((1, L, 1), lambda b: (b, 0, 0)),
        pl.BlockSpec((1, 1, L), lambda b: (b, 0, 0)),
        pl.BlockSpec((1, L, 1), lambda b: (b, 0, 0)),
        full((EMB, 8)), full((1, 8)), full((8, EMB)), full((1, EMB)),
        full((2 * EMB, 1)), full((2 * EMB, 1)), full((2 * EMB, 1)),
        full((2 * EMB, 1)), full((EMB, SIZE_M)),
        full((EMB, EMB)), full((1, EMB)), full((EMB, EMB)), full((1, EMB)),
        full((EMB, EMB)), full((1, EMB)), full((EMB, EMB)), full((1, EMB)),
    ]
    bl32 = pl.BlockSpec((1, L, SIZE_M), lambda b: (b, 0, 0))
    outs = [bl128, bl128, bl32, bl32, bl128, bl128, bl128, bl128]
    oshape = [
        jax.ShapeDtypeStruct((B, L, EMB), F32),
        jax.ShapeDtypeStruct((B, L, EMB), F32),
        jax.ShapeDtypeStruct((B, L, SIZE_M), F32),
        jax.ShapeDtypeStruct((B, L, SIZE_M), F32),
        jax.ShapeDtypeStruct((B, L, EMB), F32),
        jax.ShapeDtypeStruct((B, L, EMB), F32),
        jax.ShapeDtypeStruct((B, L, EMB), F32),
        jax.ShapeDtypeStruct((B, L, EMB), F32),
    ]
    r1 = lambda v: v.reshape(1, -1)
    return pl.pallas_call(
        _prep_body,
        grid=(B,),
        in_specs=ins,
        out_specs=outs,
        out_shape=oshape,
    )(sh, kv, scol, srow, sncol, snrow, maskc,
      p['g1_W'], r1(p['g1_b']), p['g2_W'], r1(p['g2_b']),
      p['gkh_W'], p['gvh_W'], p['gkd_W'], p['gvd_W'], MkT,
      p['eh_W'], r1(p['eh_b']), p['ah_W'], r1(p['ah_b']),
      p['ed_W'], r1(p['ed_b']), p['ad_W'], r1(p['ad_b']))


_TCH = 8


def _scan_body(mv0_ref, wh_ref, eh_ref, ah_ref, wd_ref, ed_ref, ad_ref,
               fh_ref, fd_ref, mv_s):
    @pl.when(pl.program_id(0) == 0)
    def _init():
        mv_s[...] = jnp.broadcast_to(mv0_ref[...][None, None],
                                     (2, B, SIZE_M, EMB))

    for j in range(_TCH):
        for idx, (w_ref, e_ref, a_ref, f_ref) in enumerate(
                ((wh_ref, eh_ref, ah_ref, fh_ref),
                 (wd_ref, ed_ref, ad_ref, fd_ref))):
            mv = mv_s[idx]
            wt = w_ref[:, j, :]
            et = e_ref[:, j, :]
            at = a_ref[:, j, :]
            pmat = mv * wt[:, :, None]
            f_ref[:, j, :] = jnp.sum(pmat, axis=1)
            mv_s[idx] = mv - pmat * et[:, None, :] + wt[:, :, None] * at[:, None, :]


def _scan(Mv0, wh, eh, ah, wd, ed, ad):
    n = L // _TCH
    b32 = pl.BlockSpec((B, _TCH, SIZE_M), lambda i: (0, i, 0))
    b128 = pl.BlockSpec((B, _TCH, EMB), lambda i: (0, i, 0))
    return pl.pallas_call(
        _scan_body,
        grid=(n,),
        in_specs=[pl.BlockSpec((SIZE_M, EMB), lambda i: (0, 0)),
                  b32, b128, b128, b32, b128, b128],
        out_specs=[b128, b128],
        out_shape=[jax.ShapeDtypeStruct((B, L, EMB), F32),
                   jax.ShapeDtypeStruct((B, L, EMB), F32)],
        scratch_shapes=[pltpu.VMEM((2, B, SIZE_M, EMB), F32)],
    )(Mv0, wh, eh, ah, wd, ed, ad)


def _final_body(fh_ref, fd_ref, kh_ref, kd_ref, wg_ref,
                fhw_ref, fhb_ref, fdw_ref, fdb_ref,
                w1_ref, w2_ref, wb_ref,
                ph_o, pd_o, pe_o):
    fh = fh_ref[0]
    fd = fd_ref[0]
    kh = kh_ref[0]
    kd = kd_ref[0]
    fhw = fhw_ref[...]
    fdw = fdw_ref[...]
    h = jnp.tanh(_dot(fh, fhw[:EMB]) + _dot(kh, fhw[EMB:]) + fhb_ref[...])
    d = jnp.tanh(_dot(fd, fdw[:EMB]) + _dot(kd, fdw[EMB:]) + fdb_ref[...])
    th = _sig(_dot(h, w1_ref[...]) + _dot(d, w2_ref[...]) + wb_ref[...])
    h2 = th * h
    d2 = (1.0 - th) * d
    wg = wg_ref[0]
    hc = h[:L - 1]
    dc = d[:L - 1]
    h2c = h2[:L - 1]
    d2c = d2[:L - 1]
    whd = wg[:, :EMB]
    wens = wg[:, EMB:2 * EMB]
    bhd = wg[:, 2 * EMB:2 * EMB + 1]
    bens = wg[:, 2 * EMB + 1:2 * EMB + 2]
    ph_o[0] = jnp.sum(hc * _hi(whd), axis=1, keepdims=True) + _hi(bhd)
    pd_o[0] = jnp.sum(dc * _lo(whd), axis=1, keepdims=True) + _lo(bhd)
    pe_o[0] = jnp.sum(h2c * _hi(wens), axis=1, keepdims=True) \
        + jnp.sum(d2c * _lo(wens), axis=1, keepdims=True) + _hi(bens)


def _final(fh, fd, kh, kd, Wg, p, wbrow):
    bl128 = pl.BlockSpec((1, L, EMB), lambda b: (b, 0, 0))
    full = lambda shape: pl.BlockSpec(shape, lambda b: tuple(0 for _ in shape))
    r1 = lambda v: v.reshape(1, -1)
    out1 = pl.BlockSpec((1, L - 1, 1), lambda b: (b, 0, 0))
    osh = jax.ShapeDtypeStruct((B, L - 1, 1), F32)
    return pl.pallas_call(
        _final_body,
        grid=(B,),
        in_specs=[bl128, bl128, bl128, bl128,
                  pl.BlockSpec((1, L - 1, _WCOLS), lambda b: (b, 0, 0)),
                  full((2 * EMB, EMB)), full((1, EMB)),
                  full((2 * EMB, EMB)), full((1, EMB)),
                  full((EMB, EMB)), full((EMB, EMB)), full((1, EMB))],
        out_specs=[out1, out1, out1],
        out_shape=[osh, osh, osh],
    )(fh, fd, kh, kd, Wg,
      p['fh_W'], r1(p['fh_b']), p['fd_W'], r1(p['fd_b']),
      p['w1_W'], p['w2_W'], wbrow)


def kernel(params, G, student, skill, answer):
    p = params

    answer_x = jnp.where(answer == 2, 1, answer)
    x_idx = skill + NUM_C * answer_x
    pad = lambda v: jnp.concatenate(
        [v.ravel(), jnp.zeros((_PAD - v.size,), jnp.int32)])
    idx_stu = pad(student - 1)
    idx_v = pad(x_idx)
    idx_w = pad(skill[:, 1:])

    ensWt = p['ens_W'].T
    Wpack = jnp.concatenate(
        [_packhl(p['h_W'].T, p['d_W'].T),
         _packhl(ensWt[:, :EMB], ensWt[:, EMB:]),
         _packhl(p['h_b'][:, None], p['d_b'][:, None]),
         _packhl(p['ens_b'][:, None], jnp.zeros((NUM_C, 1), F32)),
         jnp.zeros((NUM_C, _WCOLS - 2 * EMB - 2), F32)], 1)
    kv_top = jnp.concatenate(
        [p['k_emb'][:NUM_C], p['k_emb'][:NUM_C], p['k_emb'][:1]], 0)
    kv_pack = _packhl(kv_top, p['v_emb'])

    sf = skill.astype(F32)
    snf = jnp.concatenate([sf[:, 1:], jnp.full((B, 1), -1.0, F32)], 1)
    maskf = (answer != 2).astype(F32)
    scol = sf.reshape(B, L, 1)
    srow = sf.reshape(B, 1, L)
    sncol = snf.reshape(B, L, 1)
    snrow = snf.reshape(B, 1, L)
    maskc = maskf.reshape(B, L, 1)
    MkT = p['Mk'].T
    wbrow = (p['w1_b'] + p['w2_b']).reshape(1, EMB)

    g_kv, g_w = _sc_gather((EMB, _WCOLS), kv_pack, idx_v, Wpack, idx_w)

    X1 = _x1(p['stu'], p['hg_W1'], p['hg_b1'].reshape(1, EMB))
    X3 = _gmm_epi(G, X1, p['hg_W2'], p['hg_b2'].reshape(1, EMB))
    stu_emb = _gmm(G, X3)

    (g_stu,) = _sc_gather((EMB,), stu_emb, idx_stu)
    stu_h = g_stu[:B * L].reshape(B, L, EMB)
    kv = g_kv[:B * L].reshape(B, L, EMB)
    Wg = g_w[:B * (L - 1)].reshape(B, L - 1, _WCOLS)

    kh, kd, wh, wd, eh, ah, ed, ad = _prep(
        stu_h, kv, scol, srow, sncol, snrow, maskc, p, MkT)

    fh, fd = _scan(p['Mv0'], wh, eh, ah, wd, ed, ad)

    ph, pd, pe = _final(fh, fd, kh, kd, Wg, p, wbrow)
    return ph[..., 0], pd[..., 0], pe[..., 0]


# speedup vs baseline: 1.6023x; 1.0073x over previous
import functools
import jax
import jax.numpy as jnp
from jax import lax
from jax.experimental import pallas as pl
from jax.experimental.pallas import tpu as pltpu, tpu_sc as plsc

NUM_C = 2000
EMB = 128
SIZE_M = 32
NUM_STU = 2048
B = 16
L = 200
F32 = jnp.float32


def _sig(x):
    return 1.0 / (1.0 + jnp.exp(-x))


def _dot(a, b):
    return jnp.dot(a, b, preferred_element_type=F32)


def _x1_body(stu_ref, w_ref, b_ref, o_ref):
    o_ref[...] = _dot(stu_ref[...], w_ref[...]) + b_ref[...]


def _x1(stu, W1, b1row):
    return pl.pallas_call(
        _x1_body,
        out_shape=jax.ShapeDtypeStruct((NUM_STU, EMB), F32),
    )(stu, W1, b1row)


_GBLK = 256


def _gmm_epi_body(g_ref, x_ref, w2_ref, b2_ref, o_ref):
    t = _dot(g_ref[...], x_ref[...])
    o_ref[...] = _dot(jnp.maximum(t, 0.0), w2_ref[...]) + b2_ref[...]


def _gmm_body(g_ref, x_ref, o_ref):
    o_ref[...] = _dot(g_ref[...], x_ref[...])


def _gmm_epi(G, X, W2, b2row):
    n = NUM_STU // _GBLK
    return pl.pallas_call(
        _gmm_epi_body,
        grid=(n,),
        in_specs=[
            pl.BlockSpec((_GBLK, NUM_STU), lambda i: (i, 0)),
            pl.BlockSpec((NUM_STU, EMB), lambda i: (0, 0)),
            pl.BlockSpec((EMB, EMB), lambda i: (0, 0)),
            pl.BlockSpec((1, EMB), lambda i: (0, 0)),
        ],
        out_specs=pl.BlockSpec((_GBLK, EMB), lambda i: (i, 0)),
        out_shape=jax.ShapeDtypeStruct((NUM_STU, EMB), F32),
    )(G, X, W2, b2row)


def _gmm(G, X):
    n = NUM_STU // _GBLK
    return pl.pallas_call(
        _gmm_body,
        grid=(n,),
        in_specs=[
            pl.BlockSpec((_GBLK, NUM_STU), lambda i: (i, 0)),
            pl.BlockSpec((NUM_STU, EMB), lambda i: (0, 0)),
        ],
        out_specs=pl.BlockSpec((_GBLK, EMB), lambda i: (i, 0)),
        out_shape=jax.ShapeDtypeStruct((NUM_STU, EMB), F32),
    )(G, X)


_PAD = 3328
_WCOLS = 384


def _packhl(a, b):
    au = lax.bitcast_convert_type(a.astype(jnp.bfloat16), jnp.uint16)
    bu = lax.bitcast_convert_type(b.astype(jnp.bfloat16), jnp.uint16)
    w = jnp.left_shift(au.astype(jnp.uint32), 16) | bu.astype(jnp.uint32)
    return lax.bitcast_convert_type(w, F32)


def _hi(w):
    wu = lax.bitcast_convert_type(w, jnp.int32)
    return lax.bitcast_convert_type(wu & -65536, F32)


def _lo(w):
    wu = lax.bitcast_convert_type(w, jnp.int32)
    return lax.bitcast_convert_type(jnp.left_shift(wu, 16), F32)


def _sc_gather(widths, *tabs_and_idx, dtype=F32):
    n = len(widths)
    info = plsc.get_sparse_core_info()
    nc, ns = info.num_cores, info.num_subcores
    nw = nc * ns
    bpw = _PAD // nw
    mesh = plsc.VectorSubcoreMesh(core_axis_name="c", subcore_axis_name="s")

    scratch = []
    for w in widths:
        scratch.append(pltpu.VMEM((bpw,), jnp.int32))
        scratch.append(pltpu.VMEM((bpw, w), dtype))
    scratch.append(pltpu.SemaphoreType.DMA)

    @functools.partial(
        pl.kernel,
        mesh=mesh,
        out_type=[jax.ShapeDtypeStruct((_PAD, w), dtype) for w in widths],
        scratch_types=scratch,
    )
    def gather_k(*refs):
        tabs = refs[0:2 * n:2]
        idxs = refs[1:2 * n:2]
        outs = refs[2 * n:3 * n]
        ivs = refs[3 * n:3 * n + 2 * n:2]
        rvs = refs[3 * n + 1:3 * n + 2 * n:2]
        sem = refs[-1]
        wid = lax.axis_index("s") * nc + lax.axis_index("c")
        base = wid * bpw
        for th, ih, oh, iv, rv in zip(tabs, idxs, outs, ivs, rvs):
            pltpu.sync_copy(ih.at[pl.ds(base, bpw)], iv)
            pltpu.async_copy(th.at[iv], rv, sem).wait()
            pltpu.sync_copy(rv, oh.at[pl.ds(base, bpw)])

    return gather_k(*tabs_and_idx)


def _prep_body(sh_ref, kv_ref, scol_ref, srow_ref, sncol_ref,
               snrow_ref, m_ref, g1w_ref, g1b_ref, g2w_ref, g2b_ref,
               gkh_ref, gvh_ref, gkd_ref, gvd_ref, mkt_ref,
               ehw_ref, ehb_ref, ahw_ref, ahb_ref,
               edw_ref, edb_ref, adw_ref, adb_ref,
               kh_o, kd_o, wh_o, wd_o, eh_o, ah_o, ed_o, ad_o):
    sh = sh_ref[0]
    kvp = kv_ref[0]
    k0 = _hi(kvp)
    v0 = _lo(kvp)
    scol = scol_ref[0]
    srow = srow_ref[0]
    sncol = sncol_ref[0]
    snrow = snrow_ref[0]
    mcol = m_ref[0]

    emask = (snrow == scol).astype(F32)
    emask2 = (sncol == srow).astype(F32)
    deg_col = 1.0 + jnp.sum(emask, axis=1, keepdims=True)
    deg_row = 1.0 + jnp.sum(emask2, axis=0, keepdims=True)
    e_adj = emask * lax.rsqrt(deg_col) * lax.rsqrt(deg_row)
    selfco = 1.0 / deg_col
    xw1 = _dot(k0, g1w_ref[...])
    h1 = jnp.maximum(_dot(e_adj, xw1) + selfco * xw1 + g1b_ref[...], 0.0)
    xw2 = _dot(h1, g2w_ref[...])
    out2 = _dot(e_adj, xw2) + selfco * xw2 + g2b_ref[...]
    mean_h = jnp.sum(mcol * out2, axis=0, keepdims=True) * (1.0 / L)
    ash = jnp.broadcast_to(mean_h, (L, EMB))

    def gate(a, c, w_ref):
        w = w_ref[...]
        g = _sig(_dot(a, w[:EMB]) + _dot(c, w[EMB:]))
        return g * a + (1.0 - g) * c

    kh = gate(sh, k0, gkh_ref)
    vh = gate(sh, v0, gvh_ref)
    kd = gate(ash, k0, gkd_ref)
    vd = gate(ash, v0, gvd_ref)

    mkt = mkt_ref[...]

    def softmax32(x):
        m = jnp.max(x, axis=1, keepdims=True)
        ex = jnp.exp(x - m)
        return ex / jnp.sum(ex, axis=1, keepdims=True)

    kh_o[0] = kh
    kd_o[0] = kd
    wh_o[0] = softmax32(_dot(kh, mkt))
    wd_o[0] = softmax32(_dot(kd, mkt))
    eh_o[0] = _sig(_dot(vh, ehw_ref[...]) + ehb_ref[...])
    ah_o[0] = jnp.tanh(_dot(vh, ahw_ref[...]) + ahb_ref[...])
    ed_o[0] = _sig(_dot(vd, edw_ref[...]) + edb_ref[...])
    ad_o[0] = jnp.tanh(_dot(vd, adw_ref[...]) + adb_ref[...])


def _prep(sh, kv, scol, srow, sncol, snrow, maskc, p, MkT):
    bl128 = pl.BlockSpec((1, L, EMB), lambda b: (b, 0, 0))
    full = lambda shape: pl.BlockSpec(shape, lambda b: tuple(0 for _ in shape))
    ins = [
        bl128, bl128,
        pl.BlockSpec((1, L, 1), lambda b: (b, 0, 0)),
        pl.BlockSpec((1, 1, L), lambda b: (b, 0, 0)),
        pl.BlockSpec((1, L, 1), lambda b: (b, 0, 0)),
        pl.BlockSpec((1, 1, L), lambda b: (b, 0, 0)),
        pl.BlockSpec((1, L, 1), lambda b: (b, 0, 0)),
        full((EMB, 8)), full((1, 8)), full((8, EMB)), full((1, EMB)),
        full((2 * EMB, 1)), full((2 * EMB, 1)), full((2 * EMB, 1)),
        full((2 * EMB, 1)), full((EMB, SIZE_M)),
        full((EMB, EMB)), full((1, EMB)), full((EMB, EMB)), full((1, EMB)),
        full((EMB, EMB)), full((1, EMB)), full((EMB, EMB)), full((1, EMB)),
    ]
    bl32 = pl.BlockSpec((1, L, SIZE_M), lambda b: (b, 0, 0))
    outs = [bl128, bl128, bl32, bl32, bl128, bl128, bl128, bl128]
    oshape = [
        jax.ShapeDtypeStruct((B, L, EMB), F32),
        jax.ShapeDtypeStruct((B, L, EMB), F32),
        jax.ShapeDtypeStruct((B, L, SIZE_M), F32),
        jax.ShapeDtypeStruct((B, L, SIZE_M), F32),
        jax.ShapeDtypeStruct((B, L, EMB), F32),
        jax.ShapeDtypeStruct((B, L, EMB), F32),
        jax.ShapeDtypeStruct((B, L, EMB), F32),
        jax.ShapeDtypeStruct((B, L, EMB), F32),
    ]
    r1 = lambda v: v.reshape(1, -1)
    return pl.pallas_call(
        _prep_body,
        grid=(B,),
        in_specs=ins,
        out_specs=outs,
        out_shape=oshape,
    )(sh, kv, scol, srow, sncol, snrow, maskc,
      p['g1_W'], r1(p['g1_b']), p['g2_W'], r1(p['g2_b']),
      p['gkh_W'], p['gvh_W'], p['gkd_W'], p['gvd_W'], MkT,
      p['eh_W'], r1(p['eh_b']), p['ah_W'], r1(p['ah_b']),
      p['ed_W'], r1(p['ed_b']), p['ad_W'], r1(p['ad_b']))


_TCH = 40


def _scan_body(mv0_ref, wh_ref, eh_ref, ah_ref, wd_ref, ed_ref, ad_ref,
               fh_ref, fd_ref, mv_s):
    @pl.when(pl.program_id(0) == 0)
    def _init():
        mv_s[...] = jnp.broadcast_to(mv0_ref[...][None, None],
                                     (2, B, SIZE_M, EMB))

    for j in range(_TCH):
        for idx, (w_ref, e_ref, a_ref, f_ref) in enumerate(
                ((wh_ref, eh_ref, ah_ref, fh_ref),
                 (wd_ref, ed_ref, ad_ref, fd_ref))):
            mv = mv_s[idx]
            wt = w_ref[:, j, :]
            et = e_ref[:, j, :]
            at = a_ref[:, j, :]
            pmat = mv * wt[:, :, None]
            f_ref[:, j, :] = jnp.sum(pmat, axis=1)
            mv_s[idx] = mv - pmat * et[:, None, :] + wt[:, :, None] * at[:, None, :]


def _scan(Mv0, wh, eh, ah, wd, ed, ad):
    n = L // _TCH
    b32 = pl.BlockSpec((B, _TCH, SIZE_M), lambda i: (0, i, 0))
    b128 = pl.BlockSpec((B, _TCH, EMB), lambda i: (0, i, 0))
    return pl.pallas_call(
        _scan_body,
        grid=(n,),
        in_specs=[pl.BlockSpec((SIZE_M, EMB), lambda i: (0, 0)),
                  b32, b128, b128, b32, b128, b128],
        out_specs=[b128, b128],
        out_shape=[jax.ShapeDtypeStruct((B, L, EMB), F32),
                   jax.ShapeDtypeStruct((B, L, EMB), F32)],
        scratch_shapes=[pltpu.VMEM((2, B, SIZE_M, EMB), F32)],
    )(Mv0, wh, eh, ah, wd, ed, ad)


def _final_body(fh_ref, fd_ref, kh_ref, kd_ref, wg_ref,
                fhw_ref, fhb_ref, fdw_ref, fdb_ref,
                w1_ref, w2_ref, wb_ref,
                ph_o, pd_o, pe_o):
    fh = fh_ref[0]
    fd = fd_ref[0]
    kh = kh_ref[0]
    kd = kd_ref[0]
    fhw = fhw_ref[...]
    fdw = fdw_ref[...]
    h = jnp.tanh(_dot(fh, fhw[:EMB]) + _dot(kh, fhw[EMB:]) + fhb_ref[...])
    d = jnp.tanh(_dot(fd, fdw[:EMB]) + _dot(kd, fdw[EMB:]) + fdb_ref[...])
    th = _sig(_dot(h, w1_ref[...]) + _dot(d, w2_ref[...]) + wb_ref[...])
    h2 = th * h
    d2 = (1.0 - th) * d
    wg = wg_ref[0]
    hc = h[:L - 1]
    dc = d[:L - 1]
    h2c = h2[:L - 1]
    d2c = d2[:L - 1]
    whd = wg[:, :EMB]
    wens = wg[:, EMB:2 * EMB]
    bhd = wg[:, 2 * EMB:2 * EMB + 1]
    bens = wg[:, 2 * EMB + 1:2 * EMB + 2]
    ph_o[0] = jnp.sum(hc * _hi(whd), axis=1, keepdims=True) + _hi(bhd)
    pd_o[0] = jnp.sum(dc * _lo(whd), axis=1, keepdims=True) + _lo(bhd)
    pe_o[0] = jnp.sum(h2c * _hi(wens), axis=1, keepdims=True) \
        + jnp.sum(d2c * _lo(wens), axis=1, keepdims=True) + _hi(bens)


def _final(fh, fd, kh, kd, Wg, p, wbrow):
    bl128 = pl.BlockSpec((1, L, EMB), lambda b: (b, 0, 0))
    full = lambda shape: pl.BlockSpec(shape, lambda b: tuple(0 for _ in shape))
    r1 = lambda v: v.reshape(1, -1)
    out1 = pl.BlockSpec((1, L - 1, 1), lambda b: (b, 0, 0))
    osh = jax.ShapeDtypeStruct((B, L - 1, 1), F32)
    return pl.pallas_call(
        _final_body,
        grid=(B,),
        in_specs=[bl128, bl128, bl128, bl128,
                  pl.BlockSpec((1, L - 1, _WCOLS), lambda b: (b, 0, 0)),
                  full((2 * EMB, EMB)), full((1, EMB)),
                  full((2 * EMB, EMB)), full((1, EMB)),
                  full((EMB, EMB)), full((EMB, EMB)), full((1, EMB))],
        out_specs=[out1, out1, out1],
        out_shape=[osh, osh, osh],
    )(fh, fd, kh, kd, Wg,
      p['fh_W'], r1(p['fh_b']), p['fd_W'], r1(p['fd_b']),
      p['w1_W'], p['w2_W'], wbrow)


def kernel(params, G, student, skill, answer):
    p = params

    answer_x = jnp.where(answer == 2, 1, answer)
    x_idx = skill + NUM_C * answer_x
    pad = lambda v: jnp.concatenate(
        [v.ravel(), jnp.zeros((_PAD - v.size,), jnp.int32)])
    idx_stu = pad(student - 1)
    idx_v = pad(x_idx)
    idx_w = pad(skill[:, 1:])

    ensWt = p['ens_W'].T
    Wpack = jnp.concatenate(
        [_packhl(p['h_W'].T, p['d_W'].T),
         _packhl(ensWt[:, :EMB], ensWt[:, EMB:]),
         _packhl(p['h_b'][:, None], p['d_b'][:, None]),
         _packhl(p['ens_b'][:, None], jnp.zeros((NUM_C, 1), F32)),
         jnp.zeros((NUM_C, _WCOLS - 2 * EMB - 2), F32)], 1)
    kv_top = jnp.concatenate(
        [p['k_emb'][:NUM_C], p['k_emb'][:NUM_C], p['k_emb'][:1]], 0)
    kv_pack = _packhl(kv_top, p['v_emb'])

    sf = skill.astype(F32)
    snf = jnp.concatenate([sf[:, 1:], jnp.full((B, 1), -1.0, F32)], 1)
    maskf = (answer != 2).astype(F32)
    scol = sf.reshape(B, L, 1)
    srow = sf.reshape(B, 1, L)
    sncol = snf.reshape(B, L, 1)
    snrow = snf.reshape(B, 1, L)
    maskc = maskf.reshape(B, L, 1)
    MkT = p['Mk'].T
    wbrow = (p['w1_b'] + p['w2_b']).reshape(1, EMB)

    g_kv, g_w = _sc_gather((EMB, _WCOLS), kv_pack, idx_v, Wpack, idx_w)

    X1 = _x1(p['stu'], p['hg_W1'], p['hg_b1'].reshape(1, EMB))
    X3 = _gmm_epi(G, X1, p['hg_W2'], p['hg_b2'].reshape(1, EMB))
    stu_emb = _gmm(G, X3)

    (g_stu,) = _sc_gather((EMB,), stu_emb, idx_stu)
    stu_h = g_stu[:B * L].reshape(B, L, EMB)
    kv = g_kv[:B * L].reshape(B, L, EMB)
    Wg = g_w[:B * (L - 1)].reshape(B, L - 1, _WCOLS)

    kh, kd, wh, wd, eh, ah, ed, ad = _prep(
        stu_h, kv, scol, srow, sncol, snrow, maskc, p, MkT)

    fh, fd = _scan(p['Mv0'], wh, eh, ah, wd, ed, ad)

    ph, pd, pe = _final(fh, fd, kh, kd, Wg, p, wbrow)
    return ph[..., 0], pd[..., 0], pe[..., 0]
